# Initial kernel scaffold; baseline (speedup 1.0000x reference)
import jax
import jax.numpy as jnp
from jax.experimental import pallas as pl


def kernel(hidden_states, router_w, w_gate, w_up, w_down):
    raise NotImplementedError("write your pallas kernel here")


# speedup vs baseline: 1.9108x; 1.9108x over previous
import functools

import jax
import jax.numpy as jnp
from jax import lax
from jax.experimental import pallas as pl
from jax.experimental.pallas import tpu as pltpu
from jax.experimental.pallas import tpu_sc as plsc

T = 2048
H = 1024
E = 8
F = 3072
K = 2
TK = T * K
BLK = 256
NB = TK // BLK + E - 1
NBR = NB * BLK
NF = 3
FB = F // NF
NW = 32
APW = TK // NW
SUB = 64
TPW = T // NW


def _router_body(x_ref, rw_ref, logits_ref, eidx_ref, posi_ref, wn_ref, pcnt_ref):
    x = x_ref[...]
    logits = lax.dot_general(
        x, rw_ref[...], (((1,), (0,)), ((), ())),
        preferred_element_type=jnp.float32)
    logits_ref[...] = logits

    iota_e = lax.broadcasted_iota(jnp.int32, (T, E), 1)
    m1 = jnp.max(logits, axis=1, keepdims=True)
    idx1 = jnp.min(jnp.where(logits == m1, iota_e, E), axis=1, keepdims=True)
    masked = jnp.where(iota_e == idx1, -jnp.inf, logits)
    m2 = jnp.max(masked, axis=1, keepdims=True)
    idx2 = jnp.min(jnp.where(masked == m2, iota_e, E), axis=1, keepdims=True)
    ex = jnp.exp(m2 - m1)
    w1 = 1.0 / (1.0 + ex)
    w2 = ex / (1.0 + ex)
    eidx_ref[...] = jnp.concatenate([idx1, idx2], axis=1)
    wn_ref[...] = jnp.concatenate([w1, w2], axis=1)

    oh = (iota_e == idx1).astype(jnp.float32) + (iota_e == idx2).astype(jnp.float32)
    tri = (lax.broadcasted_iota(jnp.int32, (T, T), 0)
           > lax.broadcasted_iota(jnp.int32, (T, T), 1)).astype(jnp.float32)
    pref = lax.dot_general(tri, oh, (((1,), (0,)), ((), ())),
                           preferred_element_type=jnp.float32)

    offs = []
    run = jnp.int32(0)
    pcnt_rows = []
    for e in range(E):
        c_e = jnp.sum(oh[:, e:e + 1]).astype(jnp.int32)
        pad_e = ((c_e + BLK - 1) // BLK) * BLK
        offs.append(run)
        pcnt_rows.append(jnp.zeros((1, 128), jnp.int32) + pad_e)
        run = run + pad_e
    pcnt_ref[...] = jnp.concatenate(pcnt_rows, axis=0)

    offs_row = jnp.concatenate(
        [jnp.zeros((1, 1), jnp.float32) + o.astype(jnp.float32) for o in offs], axis=1)
    offs_b = jnp.zeros((T, E), jnp.float32) + offs_row
    sel1 = iota_e == idx1
    sel2 = iota_e == idx2
    pos1 = jnp.sum(jnp.where(sel1, offs_b + pref, 0.0), axis=1, keepdims=True)
    pos2 = jnp.sum(jnp.where(sel2, offs_b + pref, 0.0), axis=1, keepdims=True)
    posi_ref[...] = jnp.concatenate(
        [pos1.astype(jnp.int32), pos2.astype(jnp.int32)], axis=1)


def _router_call(x, router_w):
    return pl.pallas_call(
        _router_body,
        out_shape=(
            jax.ShapeDtypeStruct((T, E), jnp.float32),
            jax.ShapeDtypeStruct((T, K), jnp.int32),
            jax.ShapeDtypeStruct((T, K), jnp.int32),
            jax.ShapeDtypeStruct((T, K), jnp.float32),
            jax.ShapeDtypeStruct((E, 128), jnp.int32),
        ),
    )(x, router_w)


_SC_PARAMS = pltpu.CompilerParams(needs_layout_passes=False)


def _dispatch_call(x, pos_flat, w_flat):
    mesh = plsc.VectorSubcoreMesh(core_axis_name="c", subcore_axis_name="s")

    @functools.partial(
        pl.kernel, mesh=mesh, compiler_params=_SC_PARAMS,
        out_type=(
            jax.ShapeDtypeStruct((NBR, H), jnp.float32),
            jax.ShapeDtypeStruct((NBR,), jnp.float32),
        ),
        scratch_types=[
            pltpu.VMEM((SUB,), jnp.int32),
            pltpu.VMEM((SUB,), jnp.int32),
            pltpu.VMEM((SUB, H), jnp.float32),
            pltpu.VMEM((TK,), jnp.int32),
            pltpu.VMEM((TK,), jnp.float32),
            pltpu.VMEM((NBR,), jnp.float32),
            pltpu.SemaphoreType.DMA,
        ],
    )
    def k(x_hbm, posf_hbm, wf_hbm, xs_hbm, ws_hbm,
          tok_v, pos_v, rows_v, posall_v, wall_v, wsbuf_v, sem):
        wid = lax.axis_index("s") * 2 + lax.axis_index("c")
        base = wid * APW
        for sc in range(APW // SUB):
            jb = base + sc * SUB
            for v in range(SUB // 16):
                lane = lax.broadcasted_iota(jnp.int32, (16,), 0)
                tok_v[pl.ds(v * 16, 16)] = jb // K + (v * 16 + lane) // K
            pltpu.async_copy(x_hbm.at[tok_v], rows_v, sem).wait()
            pltpu.sync_copy(posf_hbm.at[pl.ds(jb, SUB)], pos_v)
            pltpu.async_copy(rows_v, xs_hbm.at[pos_v], sem).wait()

        @pl.when(wid == 0)
        def _():
            @pl.loop(0, NBR // 16)
            def _(i):
                wsbuf_v[pl.ds(i * 16, 16)] = jnp.zeros((16,), jnp.float32)

            pltpu.sync_copy(posf_hbm, posall_v)
            pltpu.sync_copy(wf_hbm, wall_v)

            @pl.loop(0, TK // 16)
            def _(i):
                idx = posall_v[pl.ds(i * 16, 16)]
                val = wall_v[pl.ds(i * 16, 16)]
                plsc.store_scatter(wsbuf_v, [idx], val)

            pltpu.sync_copy(wsbuf_v, ws_hbm)

    return k(x, pos_flat, w_flat)


def _mlp_body(bexp_ref, bval_ref, xs_ref, wg_ref, wu_ref, wd_ref, ws_ref,
              ys_ref, acc_ref):
    f = pl.program_id(0)
    b = pl.program_id(1)

    @pl.when(bval_ref[b] == 1)
    def _():
        xb = xs_ref[...].astype(jnp.bfloat16)
        g = lax.dot_general(xb, wg_ref[0].astype(jnp.bfloat16),
                            (((1,), (0,)), ((), ())),
                            preferred_element_type=jnp.float32)
        u = lax.dot_general(xb, wu_ref[0].astype(jnp.bfloat16),
                            (((1,), (0,)), ((), ())),
                            preferred_element_type=jnp.float32)
        h = (g * jax.nn.sigmoid(g) * u).astype(jnp.bfloat16)
        part = lax.dot_general(h, wd_ref[0].astype(jnp.bfloat16),
                               (((1,), (0,)), ((), ())),
                               preferred_element_type=jnp.float32)
        sl = pl.ds(b * BLK, BLK)
        prev = jnp.where(f == 0, jnp.zeros_like(part), acc_ref[sl, :])
        val = prev + part
        acc_ref[sl, :] = val

        @pl.when(f == NF - 1)
        def _():
            ys_ref[...] = val * ws_ref[...]


def _ys_index_map(f, b, be, bv):
    return jnp.where(f == NF - 1, b, NB), 0


def _mlp_call(bexp, bvalid, xs, ws2d, w_gate, w_up, w_down):
    grid_spec = pltpu.PrefetchScalarGridSpec(
        num_scalar_prefetch=2,
        grid=(NF, NB),
        in_specs=[
            pl.BlockSpec((BLK, H), lambda f, b, be, bv: (b, 0)),
            pl.BlockSpec((1, H, FB), lambda f, b, be, bv: (be[b], 0, f)),
            pl.BlockSpec((1, H, FB), lambda f, b, be, bv: (be[b], 0, f)),
            pl.BlockSpec((1, FB, H), lambda f, b, be, bv: (be[b], f, 0)),
            pl.BlockSpec((BLK, 1), lambda f, b, be, bv: (b, 0)),
        ],
        out_specs=pl.BlockSpec((BLK, H), _ys_index_map),
        scratch_shapes=[pltpu.VMEM((NBR, H), jnp.float32)],
    )
    ys_pad = pl.pallas_call(
        _mlp_body,
        grid_spec=grid_spec,
        out_shape=jax.ShapeDtypeStruct((NBR + BLK, H), jnp.float32),
        compiler_params=pltpu.CompilerParams(
            dimension_semantics=("arbitrary", "arbitrary")),
    )(bexp, bvalid, xs, w_gate, w_up, w_down, ws2d)
    return ys_pad[:NBR]


def _combine_gather_call(ys, pos_t):
    mesh = plsc.VectorSubcoreMesh(core_axis_name="c", subcore_axis_name="s")

    @functools.partial(
        pl.kernel, mesh=mesh, compiler_params=_SC_PARAMS,
        out_type=jax.ShapeDtypeStruct((K, T, H), jnp.float32),
        scratch_types=[
            pltpu.VMEM((TPW,), jnp.int32),
            pltpu.VMEM((TPW, H), jnp.float32),
            pltpu.SemaphoreType.DMA,
        ],
    )
    def k(ys_hbm, post_hbm, out_hbm, idx_v, obuf_v, sem):
        wid = lax.axis_index("s") * 2 + lax.axis_index("c")
        tb = wid * TPW
        for plane in range(K):
            pltpu.sync_copy(post_hbm.at[plane, pl.ds(tb, TPW)], idx_v)
            pltpu.async_copy(ys_hbm.at[idx_v], obuf_v, sem).wait()
            pltpu.sync_copy(obuf_v, out_hbm.at[plane, pl.ds(tb, TPW)])

    return k(ys, pos_t)


def _sum_body(p_ref, out_ref):
    out_ref[...] = p_ref[0] + p_ref[1]


_SUM_BT = 512


def _sum_call(planes):
    return pl.pallas_call(
        _sum_body,
        grid=(T // _SUM_BT,),
        in_specs=[pl.BlockSpec((K, _SUM_BT, H), lambda t: (0, t, 0))],
        out_specs=pl.BlockSpec((_SUM_BT, H), lambda t: (t, 0)),
        out_shape=jax.ShapeDtypeStruct((T, H), jnp.float32),
    )(planes)


def kernel(hidden_states, router_w, w_gate, w_up, w_down):
    x = hidden_states.reshape(T, H)
    logits, eidx, posi, wn, pcnt = _router_call(x, router_w)

    pcv = pcnt[:, 0]
    nblk = pcv // BLK
    cb = jnp.cumsum(nblk)
    bi = jnp.arange(NB, dtype=jnp.int32)
    bexp_raw = jnp.sum((bi[:, None] >= cb[None, :]).astype(jnp.int32), axis=1)
    total = cb[-1]
    last_e = jnp.max(jnp.where(nblk > 0, jnp.arange(E, dtype=jnp.int32), 0))
    bexp = jnp.where(bi < total, bexp_raw, last_e).astype(jnp.int32)
    bvalid = (bi < total).astype(jnp.int32)

    xs, ws = _dispatch_call(x, posi.reshape(-1), wn.reshape(-1))
    ys = _mlp_call(bexp, bvalid, xs, ws.reshape(NBR, 1), w_gate, w_up, w_down)
    out = _sum_call(_combine_gather_call(ys, posi.T))
    return out.reshape(hidden_states.shape), logits, eidx


# speedup vs baseline: 2.0825x; 1.0899x over previous
import functools

import jax
import jax.numpy as jnp
from jax import lax
from jax.experimental import pallas as pl
from jax.experimental.pallas import tpu as pltpu
from jax.experimental.pallas import tpu_sc as plsc

T = 2048
H = 1024
E = 8
F = 3072
K = 2
TK = T * K
BLK = 256
NB = TK // BLK + E - 1
NBR = NB * BLK
NF = 2
FB = F // NF
NW = 32
APW = TK // NW
SUB = 64
TPW = T // NW


def _router_body(x_ref, rw_ref, logits_ref, eidx_ref, posi_ref, wn_ref, pcnt_ref):
    x = x_ref[...]
    logits = lax.dot_general(
        x, rw_ref[...], (((1,), (0,)), ((), ())),
        preferred_element_type=jnp.float32)
    logits_ref[...] = logits

    iota_e = lax.broadcasted_iota(jnp.int32, (T, E), 1)
    m1 = jnp.max(logits, axis=1, keepdims=True)
    idx1 = jnp.min(jnp.where(logits == m1, iota_e, E), axis=1, keepdims=True)
    masked = jnp.where(iota_e == idx1, -jnp.inf, logits)
    m2 = jnp.max(masked, axis=1, keepdims=True)
    idx2 = jnp.min(jnp.where(masked == m2, iota_e, E), axis=1, keepdims=True)
    ex = jnp.exp(m2 - m1)
    w1 = 1.0 / (1.0 + ex)
    w2 = ex / (1.0 + ex)
    eidx_ref[...] = jnp.concatenate([idx1, idx2], axis=1)
    wn_ref[...] = jnp.concatenate([w1, w2], axis=1)

    oh = (iota_e == idx1).astype(jnp.float32) + (iota_e == idx2).astype(jnp.float32)
    tri = (lax.broadcasted_iota(jnp.int32, (T, T), 0)
           > lax.broadcasted_iota(jnp.int32, (T, T), 1)).astype(jnp.float32)
    pref = lax.dot_general(tri, oh, (((1,), (0,)), ((), ())),
                           preferred_element_type=jnp.float32)

    offs = []
    run = jnp.int32(0)
    pcnt_rows = []
    for e in range(E):
        c_e = jnp.sum(oh[:, e:e + 1]).astype(jnp.int32)
        pad_e = ((c_e + BLK - 1) // BLK) * BLK
        offs.append(run)
        pcnt_rows.append(jnp.zeros((1, 128), jnp.int32) + pad_e)
        run = run + pad_e
    pcnt_ref[...] = jnp.concatenate(pcnt_rows, axis=0)

    offs_row = jnp.concatenate(
        [jnp.zeros((1, 1), jnp.float32) + o.astype(jnp.float32) for o in offs], axis=1)
    offs_b = jnp.zeros((T, E), jnp.float32) + offs_row
    sel1 = iota_e == idx1
    sel2 = iota_e == idx2
    pos1 = jnp.sum(jnp.where(sel1, offs_b + pref, 0.0), axis=1, keepdims=True)
    pos2 = jnp.sum(jnp.where(sel2, offs_b + pref, 0.0), axis=1, keepdims=True)
    posi_ref[...] = jnp.concatenate(
        [pos1.astype(jnp.int32), pos2.astype(jnp.int32)], axis=1)


def _router_call(x, router_w):
    return pl.pallas_call(
        _router_body,
        out_shape=(
            jax.ShapeDtypeStruct((T, E), jnp.float32),
            jax.ShapeDtypeStruct((T, K), jnp.int32),
            jax.ShapeDtypeStruct((T, K), jnp.int32),
            jax.ShapeDtypeStruct((T, K), jnp.float32),
            jax.ShapeDtypeStruct((E, 128), jnp.int32),
        ),
    )(x, router_w)


_SC_PARAMS = pltpu.CompilerParams(needs_layout_passes=False)


CH = 32


def _dispatch_call(x, pos_flat):
    mesh = plsc.VectorSubcoreMesh(core_axis_name="c", subcore_axis_name="s")
    nch = APW // CH

    @functools.partial(
        pl.kernel, mesh=mesh, compiler_params=_SC_PARAMS,
        out_type=jax.ShapeDtypeStruct((NBR, H), jnp.float32),
        scratch_types=[
            pltpu.VMEM((CH,), jnp.int32),
            pltpu.VMEM((CH,), jnp.int32),
            pltpu.VMEM((CH,), jnp.int32),
            pltpu.VMEM((CH,), jnp.int32),
            pltpu.VMEM((CH, H), jnp.float32),
            pltpu.VMEM((CH, H), jnp.float32),
            pltpu.SemaphoreType.DMA,
            pltpu.SemaphoreType.DMA,
            pltpu.SemaphoreType.DMA,
            pltpu.SemaphoreType.DMA,
        ],
    )
    def k(x_hbm, posf_hbm, xs_hbm,
          tok0, tok1, pos0, pos1, buf0, buf1, gs0, gs1, ss0, ss1):
        wid = lax.axis_index("s") * 2 + lax.axis_index("c")
        base = wid * APW
        toks = (tok0, tok1)
        poss = (pos0, pos1)
        bufs = (buf0, buf1)
        gsems = (gs0, gs1)
        ssems = (ss0, ss1)

        def fill_tok(i):
            jb = base + i * CH
            for v in range(CH // 16):
                lane = lax.broadcasted_iota(jnp.int32, (16,), 0)
                toks[i % 2][pl.ds(v * 16, 16)] = jb // K + (v * 16 + lane) // K
            pltpu.sync_copy(posf_hbm.at[pl.ds(jb, CH)], poss[i % 2])
            return pltpu.async_copy(x_hbm.at[toks[i % 2]], bufs[i % 2], gsems[i % 2])

        gd = [None, None]
        sd = [None, None]
        gd[0] = fill_tok(0)
        gd[1] = fill_tok(1)
        for i in range(nch):
            p = i % 2
            gd[p].wait()
            sd[p] = pltpu.async_copy(bufs[p], xs_hbm.at[poss[p]], ssems[p])
            if i + 2 < nch:
                sd[p].wait()
                gd[p] = fill_tok(i + 2)
        sd[0].wait()
        sd[1].wait()

    return k(x, pos_flat)


def _mlp_body(bexp_ref, bval_ref, xs_ref, wg_ref, wu_ref, wd_ref,
              ys_ref, acc_ref):
    f = pl.program_id(0)
    b = pl.program_id(1)

    @pl.when(bval_ref[b] == 1)
    def _():
        xb = xs_ref[...]
        g = lax.dot_general(xb, wg_ref[0], (((1,), (0,)), ((), ())),
                            preferred_element_type=jnp.float32)
        u = lax.dot_general(xb, wu_ref[0], (((1,), (0,)), ((), ())),
                            preferred_element_type=jnp.float32)
        h = g * jax.nn.sigmoid(g) * u
        part = lax.dot_general(h, wd_ref[0], (((1,), (0,)), ((), ())),
                               preferred_element_type=jnp.float32)
        sl = pl.ds(b * BLK, BLK)

        @pl.when(f == 0)
        def _():
            acc_ref[sl, :] = part.astype(jnp.bfloat16)

        @pl.when(jnp.logical_and(f > 0, f < NF - 1))
        def _():
            acc_ref[sl, :] = (acc_ref[sl, :].astype(jnp.float32)
                              + part).astype(jnp.bfloat16)

        @pl.when(f == NF - 1)
        def _():
            ys_ref[...] = acc_ref[sl, :].astype(jnp.float32) + part


def _ys_index_map(f, b, be, bv):
    return jnp.where(f == NF - 1, b, NB), 0


def _mlp_call(bexp, bvalid, xs, w_gate, w_up, w_down):
    grid_spec = pltpu.PrefetchScalarGridSpec(
        num_scalar_prefetch=2,
        grid=(NF, NB),
        in_specs=[
            pl.BlockSpec((BLK, H), lambda f, b, be, bv: (b, 0)),
            pl.BlockSpec((1, H, FB), lambda f, b, be, bv: (be[b], 0, f)),
            pl.BlockSpec((1, H, FB), lambda f, b, be, bv: (be[b], 0, f)),
            pl.BlockSpec((1, FB, H), lambda f, b, be, bv: (be[b], f, 0)),
        ],
        out_specs=pl.BlockSpec((BLK, H), _ys_index_map),
        scratch_shapes=[pltpu.VMEM((NBR, H), jnp.bfloat16)],
    )
    ys_pad = pl.pallas_call(
        _mlp_body,
        grid_spec=grid_spec,
        out_shape=jax.ShapeDtypeStruct((NBR + BLK, H), jnp.float32),
        compiler_params=pltpu.CompilerParams(
            dimension_semantics=("arbitrary", "arbitrary")),
    )(bexp, bvalid, xs, w_gate, w_up, w_down)
    return ys_pad[:NBR]


def _combine_gather_call(ys, pos_t):
    mesh = plsc.VectorSubcoreMesh(core_axis_name="c", subcore_axis_name="s")
    nch = K * TPW // CH

    @functools.partial(
        pl.kernel, mesh=mesh, compiler_params=_SC_PARAMS,
        out_type=jax.ShapeDtypeStruct((K, T, H), jnp.float32),
        scratch_types=[
            pltpu.VMEM((CH,), jnp.int32),
            pltpu.VMEM((CH,), jnp.int32),
            pltpu.VMEM((CH, H), jnp.float32),
            pltpu.VMEM((CH, H), jnp.float32),
            pltpu.SemaphoreType.DMA,
            pltpu.SemaphoreType.DMA,
            pltpu.SemaphoreType.DMA,
            pltpu.SemaphoreType.DMA,
        ],
    )
    def k(ys_hbm, post_hbm, out_hbm, i0, i1, buf0, buf1, gs0, gs1, ws0, ws1):
        wid = lax.axis_index("s") * 2 + lax.axis_index("c")
        tb = wid * TPW
        idxs = (i0, i1)
        bufs = (buf0, buf1)
        gsems = (gs0, gs1)
        wsems = (ws0, ws1)
        npl = TPW // CH

        def chunk_loc(i):
            return i // npl, tb + (i % npl) * CH

        def start_gather(i):
            plane, rb = chunk_loc(i)
            pltpu.sync_copy(post_hbm.at[plane, pl.ds(rb, CH)], idxs[i % 2])
            return pltpu.async_copy(ys_hbm.at[idxs[i % 2]], bufs[i % 2], gsems[i % 2])

        gd = [None, None]
        wd = [None, None]
        gd[0] = start_gather(0)
        gd[1] = start_gather(1)
        for i in range(nch):
            p = i % 2
            gd[p].wait()
            plane, rb = chunk_loc(i)
            wd[p] = pltpu.async_copy(bufs[p], out_hbm.at[plane, pl.ds(rb, CH)],
                                     wsems[p])
            if i + 2 < nch:
                wd[p].wait()
                gd[p] = start_gather(i + 2)
        wd[0].wait()
        wd[1].wait()

    return k(ys, pos_t)


def _sum_body(p_ref, w_ref, out_ref):
    out_ref[...] = p_ref[0] * w_ref[0] + p_ref[1] * w_ref[1]


_SUM_BT = 512


def _sum_call(planes, wn_t):
    return pl.pallas_call(
        _sum_body,
        grid=(T // _SUM_BT,),
        in_specs=[
            pl.BlockSpec((K, _SUM_BT, H), lambda t: (0, t, 0)),
            pl.BlockSpec((K, _SUM_BT, 1), lambda t: (0, t, 0)),
        ],
        out_specs=pl.BlockSpec((_SUM_BT, H), lambda t: (t, 0)),
        out_shape=jax.ShapeDtypeStruct((T, H), jnp.float32),
    )(planes, wn_t)


def kernel(hidden_states, router_w, w_gate, w_up, w_down):
    x = hidden_states.reshape(T, H)
    logits, eidx, posi, wn, pcnt = _router_call(x, router_w)

    pcv = pcnt[:, 0]
    nblk = pcv // BLK
    cb = jnp.cumsum(nblk)
    bi = jnp.arange(NB, dtype=jnp.int32)
    bexp_raw = jnp.sum((bi[:, None] >= cb[None, :]).astype(jnp.int32), axis=1)
    total = cb[-1]
    last_e = jnp.max(jnp.where(nblk > 0, jnp.arange(E, dtype=jnp.int32), 0))
    bexp = jnp.where(bi < total, bexp_raw, last_e).astype(jnp.int32)
    bvalid = (bi < total).astype(jnp.int32)

    xs = _dispatch_call(x, posi.reshape(-1))
    ys = _mlp_call(bexp, bvalid, xs, w_gate, w_up, w_down)
    planes = _combine_gather_call(ys, posi.T)
    out = _sum_call(planes, wn.T.reshape(K, T, 1))
    return out.reshape(hidden_states.shape), logits, eidx


# speedup vs baseline: 2.1078x; 1.0121x over previous
import functools

import jax
import jax.numpy as jnp
from jax import lax
from jax.experimental import pallas as pl
from jax.experimental.pallas import tpu as pltpu
from jax.experimental.pallas import tpu_sc as plsc

T = 2048
H = 1024
E = 8
F = 3072
K = 2
TK = T * K
BLK = 256
NB = TK // BLK + E - 1
NBR = NB * BLK
NF = 2
FB = F // NF
NW = 32
APW = TK // NW
SUB = 64
TPW = T // NW


def _router_body(x_ref, rw_ref, logits_ref, eidx_ref, posi_ref, wn_ref, pcnt_ref):
    x = x_ref[...]
    logits = lax.dot_general(
        x, rw_ref[...], (((1,), (0,)), ((), ())),
        preferred_element_type=jnp.float32)
    logits_ref[...] = logits

    iota_e = lax.broadcasted_iota(jnp.int32, (T, E), 1)
    m1 = jnp.max(logits, axis=1, keepdims=True)
    idx1 = jnp.min(jnp.where(logits == m1, iota_e, E), axis=1, keepdims=True)
    masked = jnp.where(iota_e == idx1, -jnp.inf, logits)
    m2 = jnp.max(masked, axis=1, keepdims=True)
    idx2 = jnp.min(jnp.where(masked == m2, iota_e, E), axis=1, keepdims=True)
    ex = jnp.exp(m2 - m1)
    w1 = 1.0 / (1.0 + ex)
    w2 = ex / (1.0 + ex)
    eidx_ref[...] = jnp.concatenate([idx1, idx2], axis=1)
    wn_ref[...] = jnp.concatenate([w1, w2], axis=1)

    oh = (iota_e == idx1).astype(jnp.float32) + (iota_e == idx2).astype(jnp.float32)
    tri = (lax.broadcasted_iota(jnp.int32, (T, T), 0)
           > lax.broadcasted_iota(jnp.int32, (T, T), 1)).astype(jnp.float32)
    pref = lax.dot_general(tri, oh, (((1,), (0,)), ((), ())),
                           preferred_element_type=jnp.float32)

    offs = []
    run = jnp.int32(0)
    pcnt_rows = []
    for e in range(E):
        c_e = jnp.sum(oh[:, e:e + 1]).astype(jnp.int32)
        pad_e = ((c_e + BLK - 1) // BLK) * BLK
        offs.append(run)
        pcnt_rows.append(jnp.zeros((1, 128), jnp.int32) + pad_e)
        run = run + pad_e
    pcnt_ref[...] = jnp.concatenate(pcnt_rows, axis=0)

    offs_row = jnp.concatenate(
        [jnp.zeros((1, 1), jnp.float32) + o.astype(jnp.float32) for o in offs], axis=1)
    offs_b = jnp.zeros((T, E), jnp.float32) + offs_row
    sel1 = iota_e == idx1
    sel2 = iota_e == idx2
    pos1 = jnp.sum(jnp.where(sel1, offs_b + pref, 0.0), axis=1, keepdims=True)
    pos2 = jnp.sum(jnp.where(sel2, offs_b + pref, 0.0), axis=1, keepdims=True)
    posi_ref[...] = jnp.concatenate(
        [pos1.astype(jnp.int32), pos2.astype(jnp.int32)], axis=1)


def _router_call(x, router_w):
    return pl.pallas_call(
        _router_body,
        out_shape=(
            jax.ShapeDtypeStruct((T, E), jnp.float32),
            jax.ShapeDtypeStruct((T, K), jnp.int32),
            jax.ShapeDtypeStruct((T, K), jnp.int32),
            jax.ShapeDtypeStruct((T, K), jnp.float32),
            jax.ShapeDtypeStruct((E, 128), jnp.int32),
        ),
    )(x, router_w)


_SC_PARAMS = pltpu.CompilerParams(needs_layout_passes=False)


CH = 32


def _dispatch_call(x, pos_t):
    mesh = plsc.VectorSubcoreMesh(core_axis_name="c", subcore_axis_name="s")
    nch = TPW // CH

    @functools.partial(
        pl.kernel, mesh=mesh, compiler_params=_SC_PARAMS,
        out_type=jax.ShapeDtypeStruct((NBR, H), jnp.float32),
        scratch_types=[
            pltpu.VMEM((CH,), jnp.int32),
            pltpu.VMEM((CH,), jnp.int32),
            pltpu.VMEM((CH, H), jnp.float32),
            pltpu.VMEM((CH, H), jnp.float32),
            pltpu.SemaphoreType.DMA,
            pltpu.SemaphoreType.DMA,
            pltpu.SemaphoreType.DMA,
            pltpu.SemaphoreType.DMA,
        ],
    )
    def k(x_hbm, post_hbm, xs_hbm,
          pos0, pos1, buf0, buf1, gs0, gs1, ss0, ss1):
        wid = lax.axis_index("s") * 2 + lax.axis_index("c")
        base = wid * TPW
        poss = (pos0, pos1)
        bufs = (buf0, buf1)
        gsems = (gs0, gs1)
        ssems = (ss0, ss1)

        def load_rows(i):
            tb = base + i * CH
            return pltpu.async_copy(x_hbm.at[pl.ds(tb, CH)], bufs[i % 2],
                                    gsems[i % 2])

        gd = [None, None]
        gd[0] = load_rows(0)
        if nch > 1:
            gd[1] = load_rows(1)
        for i in range(nch):
            p = i % 2
            tb = base + i * CH
            gd[p].wait()
            pltpu.sync_copy(post_hbm.at[0, pl.ds(tb, CH)], poss[0])
            pltpu.sync_copy(post_hbm.at[1, pl.ds(tb, CH)], poss[1])
            d0 = pltpu.async_copy(bufs[p], xs_hbm.at[poss[0]], ssems[0])
            d1 = pltpu.async_copy(bufs[p], xs_hbm.at[poss[1]], ssems[1])
            d0.wait()
            d1.wait()
            if i + 2 < nch:
                gd[p] = load_rows(i + 2)

    return k(x, pos_t)


def _mlp_body(bexp_ref, bval_ref, xs_ref, wg_ref, wu_ref, wd_ref,
              ys_ref, acc_ref):
    f = pl.program_id(0)
    b = pl.program_id(1)

    @pl.when(bval_ref[b] == 1)
    def _():
        xb = xs_ref[...]
        g = lax.dot_general(xb, wg_ref[0], (((1,), (0,)), ((), ())),
                            preferred_element_type=jnp.float32)
        u = lax.dot_general(xb, wu_ref[0], (((1,), (0,)), ((), ())),
                            preferred_element_type=jnp.float32)
        h = g * jax.nn.sigmoid(g) * u
        part = lax.dot_general(h, wd_ref[0], (((1,), (0,)), ((), ())),
                               preferred_element_type=jnp.float32)
        sl = pl.ds(b * BLK, BLK)

        @pl.when(f == 0)
        def _():
            acc_ref[sl, :] = part.astype(jnp.bfloat16)

        @pl.when(jnp.logical_and(f > 0, f < NF - 1))
        def _():
            acc_ref[sl, :] = (acc_ref[sl, :].astype(jnp.float32)
                              + part).astype(jnp.bfloat16)

        @pl.when(f == NF - 1)
        def _():
            ys_ref[...] = acc_ref[sl, :].astype(jnp.float32) + part


def _ys_index_map(f, b, be, bv):
    return jnp.where(f == NF - 1, b, NB), 0


def _mlp_call(bexp, bvalid, xs, w_gate, w_up, w_down):
    grid_spec = pltpu.PrefetchScalarGridSpec(
        num_scalar_prefetch=2,
        grid=(NF, NB),
        in_specs=[
            pl.BlockSpec((BLK, H), lambda f, b, be, bv: (b, 0)),
            pl.BlockSpec((1, H, FB), lambda f, b, be, bv: (be[b], 0, f)),
            pl.BlockSpec((1, H, FB), lambda f, b, be, bv: (be[b], 0, f)),
            pl.BlockSpec((1, FB, H), lambda f, b, be, bv: (be[b], f, 0)),
        ],
        out_specs=pl.BlockSpec((BLK, H), _ys_index_map),
        scratch_shapes=[pltpu.VMEM((NBR, H), jnp.bfloat16)],
    )
    ys_pad = pl.pallas_call(
        _mlp_body,
        grid_spec=grid_spec,
        out_shape=jax.ShapeDtypeStruct((NBR + BLK, H), jnp.float32),
        compiler_params=pltpu.CompilerParams(
            dimension_semantics=("arbitrary", "arbitrary")),
    )(bexp, bvalid, xs, w_gate, w_up, w_down)
    return ys_pad[:NBR]


def _combine_gather_call(ys, pos_t):
    mesh = plsc.VectorSubcoreMesh(core_axis_name="c", subcore_axis_name="s")
    nch = K * TPW // CH

    @functools.partial(
        pl.kernel, mesh=mesh, compiler_params=_SC_PARAMS,
        out_type=jax.ShapeDtypeStruct((K, T, H), jnp.float32),
        scratch_types=[
            pltpu.VMEM((CH,), jnp.int32),
            pltpu.VMEM((CH,), jnp.int32),
            pltpu.VMEM((CH, H), jnp.float32),
            pltpu.VMEM((CH, H), jnp.float32),
            pltpu.SemaphoreType.DMA,
            pltpu.SemaphoreType.DMA,
            pltpu.SemaphoreType.DMA,
            pltpu.SemaphoreType.DMA,
        ],
    )
    def k(ys_hbm, post_hbm, out_hbm, i0, i1, buf0, buf1, gs0, gs1, ws0, ws1):
        wid = lax.axis_index("s") * 2 + lax.axis_index("c")
        tb = wid * TPW
        idxs = (i0, i1)
        bufs = (buf0, buf1)
        gsems = (gs0, gs1)
        wsems = (ws0, ws1)
        npl = TPW // CH

        def chunk_loc(i):
            return i // npl, tb + (i % npl) * CH

        def start_gather(i):
            plane, rb = chunk_loc(i)
            pltpu.sync_copy(post_hbm.at[plane, pl.ds(rb, CH)], idxs[i % 2])
            return pltpu.async_copy(ys_hbm.at[idxs[i % 2]], bufs[i % 2], gsems[i % 2])

        gd = [None, None]
        wd = [None, None]
        gd[0] = start_gather(0)
        gd[1] = start_gather(1)
        for i in range(nch):
            p = i % 2
            gd[p].wait()
            plane, rb = chunk_loc(i)
            wd[p] = pltpu.async_copy(bufs[p], out_hbm.at[plane, pl.ds(rb, CH)],
                                     wsems[p])
            if i + 2 < nch:
                wd[p].wait()
                gd[p] = start_gather(i + 2)
        wd[0].wait()
        wd[1].wait()

    return k(ys, pos_t)


def _sum_body(p_ref, w_ref, out_ref):
    out_ref[...] = p_ref[0] * w_ref[0] + p_ref[1] * w_ref[1]


_SUM_BT = 512


def _sum_call(planes, wn_t):
    return pl.pallas_call(
        _sum_body,
        grid=(T // _SUM_BT,),
        in_specs=[
            pl.BlockSpec((K, _SUM_BT, H), lambda t: (0, t, 0)),
            pl.BlockSpec((K, _SUM_BT, 1), lambda t: (0, t, 0)),
        ],
        out_specs=pl.BlockSpec((_SUM_BT, H), lambda t: (t, 0)),
        out_shape=jax.ShapeDtypeStruct((T, H), jnp.float32),
    )(planes, wn_t)


def kernel(hidden_states, router_w, w_gate, w_up, w_down):
    x = hidden_states.reshape(T, H)
    logits, eidx, posi, wn, pcnt = _router_call(x, router_w)

    pcv = pcnt[:, 0]
    nblk = pcv // BLK
    cb = jnp.cumsum(nblk)
    bi = jnp.arange(NB, dtype=jnp.int32)
    bexp_raw = jnp.sum((bi[:, None] >= cb[None, :]).astype(jnp.int32), axis=1)
    total = cb[-1]
    last_e = jnp.max(jnp.where(nblk > 0, jnp.arange(E, dtype=jnp.int32), 0))
    bexp = jnp.where(bi < total, bexp_raw, last_e).astype(jnp.int32)
    bvalid = (bi < total).astype(jnp.int32)

    pos_t = posi.T
    xs = _dispatch_call(x, pos_t)
    ys = _mlp_call(bexp, bvalid, xs, w_gate, w_up, w_down)
    planes = _combine_gather_call(ys, pos_t)
    out = _sum_call(planes, wn.T.reshape(K, T, 1))
    return out.reshape(hidden_states.shape), logits, eidx
